# Initial kernel scaffold; baseline (speedup 1.0000x reference)
import jax
import jax.numpy as jnp
from jax.experimental import pallas as pl


def kernel(feat, edge_index, etypes, coupling_degree, truth_value, edge_sg_ID, weight, h_bias, weight_robot_target):
    raise NotImplementedError("write your pallas kernel here")


# speedup vs baseline: 16.3071x; 16.3071x over previous
import functools

import jax
import jax.numpy as jnp
from jax import lax
from jax.experimental import pallas as pl
from jax.experimental.pallas import tpu as pltpu
from jax.experimental.pallas import tpu_sc as plsc

F32 = jnp.float32


def _prep_body(feat_ref, w_ref, hb_ref, out_ref):
    num_rels = w_ref.shape[0]
    k_dim = feat_ref.shape[1]
    out_f = w_ref.shape[2]
    row_w = k_dim * out_f
    for r in range(num_rels):
        wr = w_ref[r]
        hbr = hb_ref[pl.ds(r, 1), :]
        for k in range(k_dim):
            x = feat_ref[:, k, :]
            y = jnp.dot(x, wr, preferred_element_type=F32) + hbr
            out_ref[:, r, k * out_f:(k + 1) * out_f] = y
        out_ref[:, r, row_w:] = jnp.zeros(
            (feat_ref.shape[0], 128 - row_w), F32)


def _make_ytab(feat, weight, h_bias, bn=512):
    n, k_dim, in_f = feat.shape
    num_rels, _, out_f = weight.shape
    grid = pl.cdiv(n, bn)
    return pl.pallas_call(
        _prep_body,
        grid=(grid,),
        in_specs=[
            pl.BlockSpec((bn, k_dim, in_f), lambda i: (i, 0, 0)),
            pl.BlockSpec((num_rels, in_f, out_f), lambda i: (0, 0, 0)),
            pl.BlockSpec((num_rels, out_f), lambda i: (0, 0)),
        ],
        out_specs=pl.BlockSpec((bn, num_rels, 128), lambda i: (i, 0, 0)),
        out_shape=jax.ShapeDtypeStruct((n, num_rels, 128), F32),
    )(feat, weight, h_bias)


def _make_sc_scatter(row_w, n_workers, n_chunks, ch, hrows):
    mesh = plsc.VectorSubcoreMesh(core_axis_name="c", subcore_axis_name="s")
    rows_per_sub = hrows // 16
    nblk = rows_per_sub // ch

    n_chunks_pad = (n_chunks + 7) // 8 * 8

    @functools.partial(
        pl.kernel,
        mesh=mesh,
        out_type=jax.ShapeDtypeStruct((2, hrows, 128), F32),
        compiler_params=pltpu.CompilerParams(use_tc_tiling_on_sc=False),
        scratch_types=[
            pltpu.VMEM((n_chunks_pad, ch), jnp.int32),
            pltpu.VMEM((n_chunks_pad, ch), jnp.int32),
            pltpu.VMEM((ch,), jnp.int32),
            pltpu.VMEM((ch,), jnp.int32),
            pltpu.VMEM((ch, 128), F32),
            pltpu.VMEM((ch, row_w), F32),
            pltpu.VMEM((ch, row_w), F32),
            pltpu.VMEM((ch, 128), F32),
            pltpu.VMEM_SHARED((hrows, row_w), F32),
            pltpu.SemaphoreType.DMA,
        ],
    )
    def sc_scatter(ytab_hbm, gidx_hbm, dst_hbm, cdrow_hbm, out_hbm,
                   gidx_all, dst_all, gidx_v, dst_v, rows_v, cdr_v, msg_v,
                   wide_v, h_sh, sem):
        cid = lax.axis_index("c")
        sid = lax.axis_index("s")
        wid = cid * 16 + sid

        def zrow(j, carry):
            for c4 in range(128 // 16):
                wide_v[j, pl.ds(c4 * 16, 16)] = jnp.zeros((16,), F32)
            for c4 in range(row_w // 16):
                msg_v[j, pl.ds(c4 * 16, 16)] = jnp.zeros((16,), F32)
            return carry

        lax.fori_loop(0, ch, zrow, 0)
        for b in range(nblk):
            pltpu.sync_copy(msg_v,
                            h_sh.at[pl.ds(sid * rows_per_sub + b * ch, ch)])
        plsc.subcore_barrier()

        pltpu.sync_copy(gidx_hbm.at[wid], gidx_all)
        pltpu.sync_copy(dst_hbm.at[wid], dst_all)
        for j in range(n_chunks):
            for c4 in range(ch // 16):
                sl = pl.ds(c4 * 16, 16)
                gidx_v[sl] = gidx_all[j, sl]
                dst_v[sl] = dst_all[j, sl]
            pltpu.sync_copy(cdrow_hbm.at[wid, j], cdr_v)
            pltpu.async_copy(ytab_hbm.at[gidx_v], rows_v, sem).wait()

            def mrow(r, carry):
                for c4 in range(row_w // 16):
                    sl = pl.ds(c4 * 16, 16)
                    msg_v[r, sl] = rows_v[r, sl] * cdr_v[r, sl]
                return carry

            lax.fori_loop(0, ch, mrow, 0)
            pltpu.sync_copy(msg_v, h_sh.at[dst_v], add=True)
        plsc.subcore_barrier()

        for b in range(nblk):
            sl = pl.ds(sid * rows_per_sub + b * ch, ch)
            pltpu.sync_copy(h_sh.at[sl], msg_v)

            def crow(r, carry):
                for c4 in range(row_w // 16):
                    csl = pl.ds(c4 * 16, 16)
                    wide_v[r, csl] = msg_v[r, csl]
                return carry

            lax.fori_loop(0, ch, crow, 0)
            pltpu.sync_copy(wide_v, out_hbm.at[cid, sl])

    return sc_scatter


def _reduce_body(src_ref, dst_ref, et_ref, cd_ref, tv_ref, acc_ref):
    i = pl.program_id(0)

    @pl.when(i == 0)
    def _():
        acc_ref[...] = jnp.zeros_like(acc_ref)

    srcb = src_ref[0]
    dstb = dst_ref[0]
    etb = et_ref[0]
    b = srcb.shape[1]
    num_rels = 8
    m = ((srcb < 2) & (dstb < 2)).astype(F32)
    et2 = etb.reshape(b, 1)
    oh = (et2 == lax.broadcasted_iota(jnp.int32, (b, num_rels), 1)).astype(F32)
    tvb = tv_ref[0]
    cdb = cd_ref[0]
    for d in range(2):
        md = m * (dstb == d).astype(F32)
        c = jnp.dot(md, oh, preferred_element_type=F32)
        acc_ref[pl.ds(4 + d, 1), 0:num_rels] += c
        for s in range(2):
            wds = md * (srcb == s).astype(F32)
            t = jnp.dot(wds, tvb, preferred_element_type=F32)
            acc_ref[pl.ds(d * 2 + s, 1), 0:tvb.shape[1]] += t
            for k in range(2):
                v = jnp.dot(wds * cdb[:, k].reshape(1, b), oh,
                            preferred_element_type=F32)
                acc_ref[pl.ds(8 + d * 2 + s, 1),
                        k * num_rels:(k + 1) * num_rels] += v


def _mask_reductions(src, dst, etypes, cd, tv, blk=2000):
    e = src.shape[0]
    g = e // blk
    num_rules = tv.shape[1]
    return pl.pallas_call(
        _reduce_body,
        grid=(g,),
        in_specs=[
            pl.BlockSpec((1, 1, blk), lambda i: (i, 0, 0)),
            pl.BlockSpec((1, 1, blk), lambda i: (i, 0, 0)),
            pl.BlockSpec((1, 1, blk), lambda i: (i, 0, 0)),
            pl.BlockSpec((1, blk, 2), lambda i: (i, 0, 0)),
            pl.BlockSpec((1, blk, num_rules), lambda i: (i, 0, 0)),
        ],
        out_specs=pl.BlockSpec((16, 128), lambda i: (0, 0)),
        out_shape=jax.ShapeDtypeStruct((16, 128), F32),
    )(src.reshape(g, 1, blk), dst.reshape(g, 1, blk),
      etypes.reshape(g, 1, blk), cd.reshape(g, blk, 2),
      tv.reshape(g, blk, num_rules))


def _combine_body(hp_ref, acc_ref, hb_ref, wrt_ref, f01_ref, y16_ref, out_ref):
    i = pl.program_id(0)
    bn = hp_ref.shape[1]
    out_f = hb_ref.shape[1]
    num_rels = hb_ref.shape[0]
    num_rules = wrt_ref.shape[0]
    k_dim = out_ref.shape[1]
    row_w = k_dim * out_f
    hsum = hp_ref[0, :, 0:row_w] + hp_ref[1, :, 0:row_w]
    out_ref[...] = hsum.reshape(bn, k_dim, out_f)

    @pl.when(i == 0)
    def _():
        for d in range(2):
            for k in range(k_dim):
                tot = jnp.dot(acc_ref[pl.ds(4 + d, 1), 0:num_rels],
                              hb_ref[...], preferred_element_type=F32)
                for s in range(2):
                    f = f01_ref[pl.ds(s * k_dim + k, 1), :]
                    zrows = [jnp.dot(f, wrt_ref[u], preferred_element_type=F32)
                             for u in range(num_rules)]
                    zsk = jnp.concatenate(zrows, axis=0)
                    tot = tot + jnp.dot(acc_ref[pl.ds(d * 2 + s, 1), 0:num_rules],
                                        zsk, preferred_element_type=F32)
                    tot = tot - jnp.dot(
                        acc_ref[pl.ds(8 + d * 2 + s, 1),
                                k * num_rels:(k + 1) * num_rels],
                        y16_ref[s * num_rels:(s + 1) * num_rels,
                                k * out_f:(k + 1) * out_f],
                        preferred_element_type=F32)
                out_ref[pl.ds(d, 1), pl.ds(k, 1), :] += tot[:, None, :]


def _combine(hp, acc, h_bias, wrt, f01p, y16, n, k_dim, out_f, bn=512):
    grid = pl.cdiv(n, bn)
    num_rels, _ = h_bias.shape
    num_rules = wrt.shape[0]
    hrows = hp.shape[1]
    row_w = hp.shape[2]
    return pl.pallas_call(
        _combine_body,
        grid=(grid,),
        in_specs=[
            pl.BlockSpec((2, bn, row_w), lambda i: (0, i, 0)),
            pl.BlockSpec((16, 128), lambda i: (0, 0)),
            pl.BlockSpec((num_rels, out_f), lambda i: (0, 0)),
            pl.BlockSpec((num_rules, f01p.shape[1], out_f), lambda i: (0, 0, 0)),
            pl.BlockSpec(f01p.shape, lambda i: (0, 0)),
            pl.BlockSpec((16, row_w), lambda i: (0, 0)),
        ],
        out_specs=pl.BlockSpec((bn, k_dim, out_f), lambda i: (i, 0, 0)),
        out_shape=jax.ShapeDtypeStruct((n, k_dim, out_f), F32),
    )(hp, acc, h_bias, wrt, f01p, y16)


def kernel(feat, edge_index, etypes, coupling_degree, truth_value, edge_sg_ID,
           weight, h_bias, weight_robot_target):
    del edge_sg_ID
    n, k_dim, in_f = feat.shape
    e = etypes.shape[0]
    num_rels, _, out_f = weight.shape
    row_w = k_dim * out_f

    src = edge_index[0]
    dst = edge_index[1]

    ytab = _make_ytab(feat, weight, h_bias)
    ytab_flat = ytab.reshape(n * num_rels, 128)

    n_workers, ch = 32, 128
    n_chunks = pl.cdiv(e, n_workers * ch)
    ep = n_workers * ch * n_chunks
    pad = ep - e
    hrows = ((n + 16 * ch - 1) // (16 * ch)) * (16 * ch)

    n_chunks_pad = (n_chunks + 7) // 8 * 8
    chunk_pad = ((0, 0), (0, n_chunks_pad - n_chunks), (0, 0))

    gidx = src * num_rels + etypes
    gidx_p = jnp.pad(
        jnp.concatenate([gidx, jnp.zeros((pad,), jnp.int32)]).reshape(
            n_workers, n_chunks, ch), chunk_pad)
    dst_p = jnp.pad(
        jnp.concatenate([dst, jnp.full((pad,), n, jnp.int32)]).reshape(
            n_workers, n_chunks, ch), chunk_pad, constant_values=n)
    cd_p = jnp.concatenate(
        [coupling_degree, jnp.zeros((pad, k_dim), F32)], axis=0)
    cdrow = jnp.repeat(cd_p, out_f, axis=1)

    sc_fn = _make_sc_scatter(row_w, n_workers, n_chunks, ch, hrows)
    hp = sc_fn(ytab_flat, gidx_p, dst_p,
               cdrow.reshape(n_workers, n_chunks, ch, row_w))

    acc = _mask_reductions(src, dst, etypes, coupling_degree, truth_value)

    f01p = jnp.zeros((8, in_f), F32).at[0:2 * k_dim].set(
        feat[0:2].reshape(2 * k_dim, in_f))
    y16 = ytab_flat[0:2 * num_rels, 0:row_w]
    return _combine(hp, acc, h_bias, weight_robot_target, f01p, y16,
                    n, k_dim, out_f)
